# baseline (speedup 1.0000x reference)
import dataclasses
import functools

import jax
import jax.numpy as jnp
from jax import lax
from jax.experimental import pallas as pl
from jax.experimental.pallas import tpu as pltpu
from jax.experimental.pallas import tpu_sc as plsc

B = 4096
S = 200
D = 64
L = 16
NC = 2
NS = 16
NW = NC * NS
RPW = B // NW
SPAD = ((S + L - 1) // L) * L
NG = SPAD // L
C1 = 128
C2 = S - C1
FLAT = RPW * S


def _sc_body(embed_hbm, short_hbm, weight_hbm, bias_hbm, out_hbm,
             idx_v, embed_v, rows0, rows1, bv0, bv1, out_v,
             sem_in, sem_g0, sem_g1):
    wid = lax.axis_index("s") * NC + lax.axis_index("c")
    row0 = wid * RPW
    base = wid * FLAT

    cp_i = pltpu.async_copy(short_hbm.at[pl.ds(base, FLAT)], idx_v, sem_in)
    cp_e = pltpu.async_copy(embed_hbm.at[pl.ds(row0, RPW)], embed_v, sem_in)
    cp_i.wait()
    cp_e.wait()

    def fire(r, rows_v, bv, sem):
        off = r * S
        pltpu.async_copy(weight_hbm.at[idx_v.at[pl.ds(off, C1)]],
                         rows_v.at[pl.ds(0, C1)], sem)
        pltpu.async_copy(weight_hbm.at[idx_v.at[pl.ds(off + C1, C2)]],
                         rows_v.at[pl.ds(C1, C2)], sem)
        pltpu.async_copy(bias_hbm.at[idx_v.at[pl.ds(off, C1)]],
                         bv.at[pl.ds(0, C1)], sem)
        pltpu.async_copy(bias_hbm.at[idx_v.at[pl.ds(off + C1, C2)]],
                         bv.at[pl.ds(C1, C2)], sem)

    def drain(r, rows_v, bv, sem):
        off = r * S
        pltpu.make_async_copy(weight_hbm.at[idx_v.at[pl.ds(off, C1)]],
                              rows_v.at[pl.ds(0, C1)], sem).wait()
        pltpu.make_async_copy(weight_hbm.at[idx_v.at[pl.ds(off + C1, C2)]],
                              rows_v.at[pl.ds(C1, C2)], sem).wait()
        pltpu.make_async_copy(bias_hbm.at[idx_v.at[pl.ds(off, C1)]],
                              bv.at[pl.ds(0, C1)], sem).wait()
        pltpu.make_async_copy(bias_hbm.at[idx_v.at[pl.ds(off + C1, C2)]],
                              bv.at[pl.ds(C1, C2)], sem).wait()

    row_ids = [lax.iota(jnp.int32, L) + g * L for g in range(NG)]

    def compute(r, rows_v, bv):
        r_vec = jnp.full((L,), r, jnp.int32)

        def dbody(d, accs):
            cols = jnp.full((L,), d, jnp.int32)
            e_d = plsc.load_gather(embed_v, [r_vec, cols])
            return tuple(
                accs[g] + plsc.load_gather(rows_v, [row_ids[g], cols]) * e_d
                for g in range(NG))

        accs = lax.fori_loop(
            0, D, dbody, tuple(jnp.zeros((L,), jnp.float32) for _ in range(NG)))
        out_off = r * S
        for g in range(NG):
            out_v[pl.ds(out_off + g * L, L)] = accs[g] + bv[pl.ds(g * L, L)]

    fire(0, rows0, bv0, sem_g0)

    @pl.loop(0, RPW // 2)
    def _(p):
        r0 = p * 2
        r1 = r0 + 1
        fire(r1, rows1, bv1, sem_g1)
        drain(r0, rows0, bv0, sem_g0)
        compute(r0, rows0, bv0)

        @pl.when(p < RPW // 2 - 1)
        def _():
            fire(r0 + 2, rows0, bv0, sem_g0)

        drain(r1, rows1, bv1, sem_g1)
        compute(r1, rows1, bv1)

    pltpu.sync_copy(out_v.at[pl.ds(0, FLAT)], out_hbm.at[pl.ds(base, FLAT)])


_cp = pltpu.CompilerParams()
for _field, _val in (("needs_layout_passes", False),
                     ("use_tc_tiling_on_sc", False)):
    if _field in pltpu.CompilerParams.__dataclass_fields__:
        _cp = dataclasses.replace(_cp, **{_field: _val})


@functools.partial(
    pl.kernel,
    out_type=jax.ShapeDtypeStruct((B * S,), jnp.float32),
    mesh=plsc.VectorSubcoreMesh(core_axis_name="c", subcore_axis_name="s"),
    compiler_params=_cp,
    scratch_types=[
        pltpu.VMEM((FLAT,), jnp.int32),
        pltpu.VMEM((RPW, D), jnp.float32),
        pltpu.VMEM((SPAD, D), jnp.float32),
        pltpu.VMEM((SPAD, D), jnp.float32),
        pltpu.VMEM((SPAD,), jnp.float32),
        pltpu.VMEM((SPAD,), jnp.float32),
        pltpu.VMEM((FLAT + 8,), jnp.float32),
        pltpu.SemaphoreType.DMA,
        pltpu.SemaphoreType.DMA,
        pltpu.SemaphoreType.DMA,
    ],
)
def _sc_sparse_linear(embed_hbm, short_hbm, weight_hbm, bias_hbm, out_hbm,
                      idx_v, embed_v, rows0, rows1, bv0, bv1, out_v,
                      sem_in, sem_g0, sem_g1):
    _sc_body(embed_hbm, short_hbm, weight_hbm, bias_hbm, out_hbm,
             idx_v, embed_v, rows0, rows1, bv0, bv1, out_v,
             sem_in, sem_g0, sem_g1)


@jax.jit
def kernel(embed, shortlist, weight, bias):
    b, s = shortlist.shape
    short_flat = shortlist.astype(jnp.int32).reshape(-1)
    bias_flat = bias.reshape(-1)
    out = _sc_sparse_linear(embed, short_flat, weight, bias_flat)
    return out.reshape(b, s)
